# baseline (speedup 1.0000x reference)
import functools

import jax
import jax.numpy as jnp
from jax import lax
from jax.experimental import pallas as pl
from jax.experimental.pallas import tpu as pltpu
from jax.experimental.pallas import tpu_sc as plsc

N = 10000
E = 320000
F = 128

NC = 2
NS = 16
NW = NC * NS
CHUNK = 128
K0 = 160
DBLK = 16
NCHUNKS = NS * K0
NCPAD = NCHUNKS
EPAD = NCHUNKS * CHUNK
NACC = 10112
RPA = NACC // NS


def _mesh():
    return plsc.VectorSubcoreMesh(
        core_axis_name="c", subcore_axis_name="s", num_cores=1)


@functools.partial(
    pl.kernel,
    out_type=jax.ShapeDtypeStruct((1, NACC, F), jnp.float32),
    mesh=_mesh(),
    scratch_types=[
        pltpu.VMEM((2, DBLK, CHUNK), jnp.int32),
        pltpu.VMEM((DBLK, CHUNK), jnp.int32),
        pltpu.VMEM((2, CHUNK, F), jnp.float32),
        pltpu.VMEM_SHARED((NACC, F), jnp.float32),
        pltpu.SemaphoreType.DMA,
        pltpu.SemaphoreType.DMA,
        pltpu.SemaphoreType.DMA,
    ],
)
def _sc_aggregate(g_hbm, src_hbm, dst_hbm, zeros_hbm, out_hbm,
                  src_v, dst_v, rows_v, acc, gsem, ssem, psem):
    cid = lax.axis_index("c")
    sid = lax.axis_index("s")
    r0 = sid * RPA
    roff = sid * K0
    kc = K0
    NBLK = K0 // DBLK
    pltpu.sync_copy(zeros_hbm.at[pl.ds(r0, RPA)], acc.at[pl.ds(r0, RPA)])
    pltpu.sync_copy(src_hbm.at[pl.ds(roff, DBLK)], src_v.at[0])
    plsc.subcore_barrier()

    pltpu.async_copy(g_hbm.at[src_v.at[0, 0]], rows_v.at[0], gsem)

    def blk_body(bi, _):
        slot = lax.rem(bi, 2)

        @pl.when(bi + 1 < NBLK)
        def _():
            pltpu.async_copy(
                src_hbm.at[pl.ds(roff + (bi + 1) * DBLK, DBLK)],
                src_v.at[1 - slot], psem)

        @pl.when(bi >= 1)
        def _():
            pltpu.make_async_copy(
                rows_v.at[0], acc.at[dst_v.at[0]], ssem).wait()

        pltpu.sync_copy(dst_hbm.at[pl.ds(roff + bi * DBLK, DBLK)], dst_v)

        def body(jj, _):
            b = lax.rem(bi * DBLK + jj, 2)
            pltpu.make_async_copy(
                g_hbm.at[src_v.at[slot, jj]], rows_v.at[b], gsem).wait()

            @pl.when(jj >= 1)
            def _():
                pltpu.make_async_copy(
                    rows_v.at[1 - b], acc.at[dst_v.at[jj]], ssem).wait()

            @pl.when(jj + 1 < DBLK)
            def _():
                pltpu.async_copy(
                    g_hbm.at[src_v.at[slot, jj + 1]], rows_v.at[1 - b],
                    gsem)

            pltpu.async_copy(
                rows_v.at[b], acc.at[dst_v.at[jj]], ssem, add=True)
            return 0

        lax.fori_loop(0, DBLK, body, 0)

        @pl.when(bi + 1 < NBLK)
        def _():
            pltpu.make_async_copy(
                src_hbm.at[pl.ds(roff + (bi + 1) * DBLK, DBLK)],
                src_v.at[1 - slot], psem).wait()
            pltpu.async_copy(
                g_hbm.at[src_v.at[1 - slot, 0]],
                rows_v.at[lax.rem((bi + 1) * DBLK, 2)], gsem)

        return 0

    lax.fori_loop(0, NBLK, blk_body, 0)
    pltpu.make_async_copy(rows_v.at[0], acc.at[dst_v.at[0]], ssem).wait()
    plsc.subcore_barrier()
    pltpu.sync_copy(acc.at[pl.ds(r0, RPA)], out_hbm.at[0, pl.ds(r0, RPA)])


_BR = 1000


def _tc_prep_body(dega_ref, feat_ref, norm_ref, inv1_ref,
                  g_ref, s_ref):
    deg = dega_ref[0, :, 0:1]
    deg = jnp.maximum(deg, 1.0)
    norm = jax.lax.rsqrt(deg)
    inv1 = 1.0 / (deg + 1.0)
    normb = jnp.broadcast_to(norm, (_BR, F))
    inv1b = jnp.broadcast_to(inv1, (_BR, F))
    f = feat_ref[...]
    norm_ref[...] = normb
    inv1_ref[...] = inv1b
    g_ref[...] = f * normb
    s_ref[...] = f * inv1b


def _tc_prep(degp, feat):
    grid = (N // _BR,)
    return pl.pallas_call(
        _tc_prep_body,
        grid=grid,
        in_specs=[
            pl.BlockSpec((1, _BR, F), lambda i: (0, i, 0)),
            pl.BlockSpec((_BR, F), lambda i: (i, 0)),
        ],
        out_specs=[pl.BlockSpec((_BR, F), lambda i: (i, 0))] * 4,
        out_shape=[jax.ShapeDtypeStruct((N, F), jnp.float32)] * 4,
    )(degp, feat)


def _tc_combine_body(agga_ref, norm_ref, inv1_ref, s_ref,
                     g_ref, s2_ref):
    agg = agga_ref[0]
    normb = norm_ref[...]
    f = agg * normb + s_ref[...]
    g_ref[...] = f * normb
    s2_ref[...] = f * inv1_ref[...]


def _tc_combine(aggp, normb, inv1b, s):
    grid = (N // _BR,)
    return pl.pallas_call(
        _tc_combine_body,
        grid=grid,
        in_specs=[
            pl.BlockSpec((1, _BR, F), lambda i: (0, i, 0)),
        ] + [pl.BlockSpec((_BR, F), lambda i: (i, 0))] * 3,
        out_specs=[pl.BlockSpec((_BR, F), lambda i: (i, 0))] * 2,
        out_shape=[jax.ShapeDtypeStruct((N, F), jnp.float32)] * 2,
    )(aggp, normb, inv1b, s)


def _tc_final_body(agga_ref, norm_ref, s_ref, wt_ref, b_ref,
                   out_ref):
    agg = agga_ref[0]
    f = agg * norm_ref[...] + s_ref[...]
    out_ref[...] = (
        jnp.dot(f, wt_ref[...], preferred_element_type=jnp.float32)
        + b_ref[...]
    )


def _tc_final(aggp, normb, s, wt, b2):
    grid = (N // _BR,)
    return pl.pallas_call(
        _tc_final_body,
        grid=grid,
        in_specs=[
            pl.BlockSpec((1, _BR, F), lambda i: (0, i, 0)),
            pl.BlockSpec((_BR, F), lambda i: (i, 0)),
            pl.BlockSpec((_BR, F), lambda i: (i, 0)),
            pl.BlockSpec((F, F), lambda i: (0, 0)),
            pl.BlockSpec((1, F), lambda i: (0, 0)),
        ],
        out_specs=pl.BlockSpec((_BR, F), lambda i: (i, 0)),
        out_shape=jax.ShapeDtypeStruct((N, F), jnp.float32),
    )(aggp, normb, s, wt, b2)


def kernel(feat, edge_index, W, b):
    src = edge_index[0]
    dst = edge_index[1]
    srcr = jnp.concatenate(
        [src, jnp.zeros((NCPAD * CHUNK - E,), jnp.int32)]).reshape(
            NCPAD, CHUNK)
    dstr = jnp.concatenate(
        [dst, jnp.full((NCPAD * CHUNK - E,), N, jnp.int32)]).reshape(
            NCPAD, CHUNK)

    zeros_f = jnp.zeros((NACC, F), jnp.float32)
    ones_t = jnp.ones((N, F), jnp.float32)

    seqr = (jnp.arange(NCPAD * CHUNK, dtype=jnp.int32) % N).reshape(
        NCPAD, CHUNK)
    degp = _sc_aggregate(ones_t, seqr, dstr, zeros_f)
    normb, inv1b, g0, s0 = _tc_prep(degp, feat)
    agg0 = _sc_aggregate(g0, srcr, dstr, zeros_f)
    g1, s1 = _tc_combine(agg0, normb, inv1b, s0)
    agg1 = _sc_aggregate(g1, srcr, dstr, zeros_f)
    out = _tc_final(agg1, normb, s1, W.T, b.reshape(1, F))
    return out


# speedup vs baseline: 1.2156x; 1.2156x over previous
import functools

import jax
import jax.numpy as jnp
from jax import lax
from jax.experimental import pallas as pl
from jax.experimental.pallas import tpu as pltpu
from jax.experimental.pallas import tpu_sc as plsc

N = 10000
E = 320000
F = 128

NC = 2
NS = 16
NW = NC * NS
CHUNK = 128
K0 = 128
K1 = 32
DBLK = 8
NCHUNKS = NS * (K0 + K1)
NCPAD = NCHUNKS + K0 - K1
EPAD = NCHUNKS * CHUNK
NACC = 10112
RPA = NACC // NS


def _mesh():
    return plsc.VectorSubcoreMesh(core_axis_name="c", subcore_axis_name="s")


@functools.partial(
    pl.kernel,
    out_type=jax.ShapeDtypeStruct((NC, NACC, F), jnp.float32),
    mesh=_mesh(),
    scratch_types=[
        pltpu.VMEM((K0, CHUNK), jnp.int32),
        pltpu.VMEM((DBLK, CHUNK), jnp.int32),
        pltpu.VMEM((2, CHUNK, F), jnp.float32),
        pltpu.VMEM_SHARED((NACC, F), jnp.float32),
        pltpu.SemaphoreType.DMA,
        pltpu.SemaphoreType.DMA,
    ],
)
def _sc_aggregate(g_hbm, src_hbm, dst_hbm, zeros_hbm, out_hbm,
                  src_v, dst_v, rows_v, acc, gsem, ssem):
    cid = lax.axis_index("c")
    sid = lax.axis_index("s")
    r0 = sid * RPA
    roff = jnp.where(cid == 0, sid * K0, NS * K0 + sid * K1)
    kc = jnp.where(cid == 0, K0, K1)
    pltpu.sync_copy(zeros_hbm.at[pl.ds(r0, RPA)], acc.at[pl.ds(r0, RPA)])
    pltpu.sync_copy(src_hbm.at[pl.ds(roff, K0)], src_v)
    plsc.subcore_barrier()

    pltpu.async_copy(g_hbm.at[src_v.at[0]], rows_v.at[0], gsem)

    def blk_body(bi, _):
        @pl.when(bi >= 1)
        def _():
            pltpu.make_async_copy(
                rows_v.at[0], acc.at[dst_v.at[0]], ssem).wait()

        pltpu.sync_copy(dst_hbm.at[pl.ds(roff + bi * DBLK, DBLK)], dst_v)

        def body(jj, _):
            j = bi * DBLK + jj
            b = lax.rem(j, 2)
            pltpu.make_async_copy(
                g_hbm.at[src_v.at[j]], rows_v.at[b], gsem).wait()

            @pl.when(jj >= 1)
            def _():
                pltpu.make_async_copy(
                    rows_v.at[1 - b], acc.at[dst_v.at[jj]], ssem).wait()

            @pl.when(j + 1 < kc)
            def _():
                pltpu.async_copy(
                    g_hbm.at[src_v.at[j + 1]], rows_v.at[1 - b], gsem)

            pltpu.async_copy(
                rows_v.at[b], acc.at[dst_v.at[jj]], ssem, add=True)
            return 0

        lax.fori_loop(0, DBLK, body, 0)
        return 0

    lax.fori_loop(0, kc // DBLK, blk_body, 0)
    pltpu.make_async_copy(rows_v.at[0], acc.at[dst_v.at[0]], ssem).wait()
    plsc.subcore_barrier()
    pltpu.sync_copy(acc.at[pl.ds(r0, RPA)], out_hbm.at[cid, pl.ds(r0, RPA)])


_BR = 1000


def _tc_prep_body(dega_ref, degb_ref, feat_ref, norm_ref, inv1_ref,
                  g_ref, s_ref):
    deg = dega_ref[0, :, 0:1] + degb_ref[0, :, 0:1]
    deg = jnp.maximum(deg, 1.0)
    norm = jax.lax.rsqrt(deg)
    inv1 = 1.0 / (deg + 1.0)
    normb = jnp.broadcast_to(norm, (_BR, F))
    inv1b = jnp.broadcast_to(inv1, (_BR, F))
    f = feat_ref[...]
    norm_ref[...] = normb
    inv1_ref[...] = inv1b
    g_ref[...] = f * normb
    s_ref[...] = f * inv1b


def _tc_prep(degp, feat):
    grid = (N // _BR,)
    return pl.pallas_call(
        _tc_prep_body,
        grid=grid,
        in_specs=[
            pl.BlockSpec((1, _BR, F), lambda i: (0, i, 0)),
            pl.BlockSpec((1, _BR, F), lambda i: (1, i, 0)),
            pl.BlockSpec((_BR, F), lambda i: (i, 0)),
        ],
        out_specs=[pl.BlockSpec((_BR, F), lambda i: (i, 0))] * 4,
        out_shape=[jax.ShapeDtypeStruct((N, F), jnp.float32)] * 4,
    )(degp, degp, feat)


def _tc_combine_body(agga_ref, aggb_ref, norm_ref, inv1_ref, s_ref,
                     g_ref, s2_ref):
    agg = agga_ref[0] + aggb_ref[0]
    normb = norm_ref[...]
    f = agg * normb + s_ref[...]
    g_ref[...] = f * normb
    s2_ref[...] = f * inv1_ref[...]


def _tc_combine(aggp, normb, inv1b, s):
    grid = (N // _BR,)
    return pl.pallas_call(
        _tc_combine_body,
        grid=grid,
        in_specs=[
            pl.BlockSpec((1, _BR, F), lambda i: (0, i, 0)),
            pl.BlockSpec((1, _BR, F), lambda i: (1, i, 0)),
        ] + [pl.BlockSpec((_BR, F), lambda i: (i, 0))] * 3,
        out_specs=[pl.BlockSpec((_BR, F), lambda i: (i, 0))] * 2,
        out_shape=[jax.ShapeDtypeStruct((N, F), jnp.float32)] * 2,
    )(aggp, aggp, normb, inv1b, s)


def _tc_final_body(agga_ref, aggb_ref, norm_ref, s_ref, wt_ref, b_ref,
                   out_ref):
    agg = agga_ref[0] + aggb_ref[0]
    f = agg * norm_ref[...] + s_ref[...]
    out_ref[...] = (
        jnp.dot(f, wt_ref[...], preferred_element_type=jnp.float32)
        + b_ref[...]
    )


def _tc_final(aggp, normb, s, wt, b2):
    grid = (N // _BR,)
    return pl.pallas_call(
        _tc_final_body,
        grid=grid,
        in_specs=[
            pl.BlockSpec((1, _BR, F), lambda i: (0, i, 0)),
            pl.BlockSpec((1, _BR, F), lambda i: (1, i, 0)),
            pl.BlockSpec((_BR, F), lambda i: (i, 0)),
            pl.BlockSpec((_BR, F), lambda i: (i, 0)),
            pl.BlockSpec((F, F), lambda i: (0, 0)),
            pl.BlockSpec((1, F), lambda i: (0, 0)),
        ],
        out_specs=pl.BlockSpec((_BR, F), lambda i: (i, 0)),
        out_shape=jax.ShapeDtypeStruct((N, F), jnp.float32),
    )(aggp, aggp, normb, s, wt, b2)


def kernel(feat, edge_index, W, b):
    src = edge_index[0]
    dst = edge_index[1]
    srcr = jnp.concatenate(
        [src, jnp.zeros((NCPAD * CHUNK - E,), jnp.int32)]).reshape(
            NCPAD, CHUNK)
    dstr = jnp.concatenate(
        [dst, jnp.full((NCPAD * CHUNK - E,), N, jnp.int32)]).reshape(
            NCPAD, CHUNK)

    zeros_f = jnp.zeros((NACC, F), jnp.float32)
    ones_t = jnp.ones((N, F), jnp.float32)

    seqr = (jnp.arange(NCPAD * CHUNK, dtype=jnp.int32) % N).reshape(
        NCPAD, CHUNK)
    degp = _sc_aggregate(ones_t, seqr, dstr, zeros_f)
    normb, inv1b, g0, s0 = _tc_prep(degp, feat)
    agg0 = _sc_aggregate(g0, srcr, dstr, zeros_f)
    g1, s1 = _tc_combine(agg0, normb, inv1b, s0)
    agg1 = _sc_aggregate(g1, srcr, dstr, zeros_f)
    out = _tc_final(agg1, normb, s1, W.T, b.reshape(1, F))
    return out
